# Initial kernel scaffold; baseline (speedup 1.0000x reference)
import jax
import jax.numpy as jnp
from jax.experimental import pallas as pl


def kernel(tokens, embedding):
    raise NotImplementedError("write your pallas kernel here")


# speedup vs baseline: 1.4587x; 1.4587x over previous
import functools

import jax
import jax.numpy as jnp
from jax import lax
from jax.experimental import pallas as pl
from jax.experimental.pallas import tpu as pltpu
from jax.experimental.pallas import tpu_sc as plsc

VOCAB_ = 1000000
SEQ_ = 200
BATCH_ = 4096
EMB_ = 32

NUM_WORKERS = 32
TOTAL = BATCH_ * SEQ_
PER_WORKER = TOTAL // NUM_WORKERS
CHUNK = 1024
NUM_CHUNKS = PER_WORKER // CHUNK


def _sc_gather(idx_flat, embedding):
  mesh = plsc.VectorSubcoreMesh(core_axis_name="c", subcore_axis_name="s")

  @functools.partial(
      pl.kernel,
      out_type=jax.ShapeDtypeStruct((TOTAL, EMB_), jnp.float32),
      mesh=mesh,
      scratch_types=[
          pltpu.VMEM((CHUNK,), jnp.int32),
          pltpu.VMEM((CHUNK, EMB_), jnp.float32),
          pltpu.SemaphoreType.DMA,
      ],
      compiler_params=pltpu.CompilerParams(use_tc_tiling_on_sc=False),
  )
  def body(idx_hbm, table_hbm, out_hbm, idx_v, rows_v, sem):
    wid = lax.axis_index("s") * 2 + lax.axis_index("c")
    base = wid * PER_WORKER

    def step(i, carry):
      off = base + i * CHUNK
      pltpu.sync_copy(idx_hbm.at[pl.ds(off, CHUNK)], idx_v)
      pltpu.async_copy(table_hbm.at[idx_v], rows_v, sem).wait()
      pltpu.sync_copy(rows_v, out_hbm.at[pl.ds(off, CHUNK)])
      return carry

    lax.fori_loop(0, NUM_CHUNKS, step, 0)

  return body(idx_flat, embedding)


def kernel(tokens, embedding):
  idx_flat = tokens.astype(jnp.int32).reshape(TOTAL)
  out = _sc_gather(idx_flat, embedding)
  return out.reshape(BATCH_, SEQ_, EMB_)


# speedup vs baseline: 1.5004x; 1.0286x over previous
import functools

import jax
import jax.numpy as jnp
from jax import lax
from jax.experimental import pallas as pl
from jax.experimental.pallas import tpu as pltpu
from jax.experimental.pallas import tpu_sc as plsc

VOCAB_ = 1000000
SEQ_ = 200
BATCH_ = 4096
EMB_ = 32

NUM_WORKERS = 32
TOTAL = BATCH_ * SEQ_
PER_WORKER = TOTAL // NUM_WORKERS
CHUNK = 1280
NUM_CHUNKS = PER_WORKER // CHUNK
NBUF = 2


def _sc_gather(idx_flat, embedding):
  mesh = plsc.VectorSubcoreMesh(core_axis_name="c", subcore_axis_name="s")

  @functools.partial(
      pl.kernel,
      out_type=jax.ShapeDtypeStruct((TOTAL, EMB_), jnp.float32),
      mesh=mesh,
      scratch_types=[
          pltpu.VMEM((PER_WORKER,), jnp.int32),
          pltpu.VMEM((CHUNK, EMB_), jnp.float32),
          pltpu.VMEM((CHUNK, EMB_), jnp.float32),
          pltpu.SemaphoreType.DMA,
          pltpu.SemaphoreType.DMA,
          pltpu.SemaphoreType.DMA,
          pltpu.SemaphoreType.DMA,
      ],
      compiler_params=pltpu.CompilerParams(use_tc_tiling_on_sc=False),
  )
  def body(idx_hbm, table_hbm, out_hbm, idx_all, rows0, rows1, g0, g1, w0,
           w1):
    rows = (rows0, rows1)
    gsem = (g0, g1)
    wsem = (w0, w1)
    wid = lax.axis_index("s") * 2 + lax.axis_index("c")
    base = wid * PER_WORKER

    pltpu.sync_copy(idx_hbm.at[pl.ds(base, PER_WORKER)], idx_all)

    def idx_ref(i):
      return idx_all.at[pl.ds(i * CHUNK, CHUNK)]

    def out_ref(i):
      return out_hbm.at[pl.ds(base + i * CHUNK, CHUNK)]

    for b in range(NBUF):
      pltpu.async_copy(table_hbm.at[idx_ref(b)], rows[b], gsem[b])

    def step(it, carry):
      g = it * NBUF
      for b in range(NBUF):
        i = g + b
        pltpu.make_async_copy(table_hbm.at[idx_ref(i)], rows[b],
                              gsem[b]).wait()
        pltpu.async_copy(rows[b], out_ref(i), wsem[b])
        j = i + NBUF

        @pl.when(j < NUM_CHUNKS)
        def _():
          pltpu.make_async_copy(rows[b], out_ref(i), wsem[b]).wait()
          pltpu.async_copy(table_hbm.at[idx_ref(j)], rows[b], gsem[b])

      return carry

    lax.fori_loop(0, NUM_CHUNKS // NBUF, step, 0)
    for b in range(NBUF):
      i = NUM_CHUNKS - NBUF + b
      pltpu.make_async_copy(rows[b], out_ref(i), wsem[b]).wait()

  return body(idx_flat, embedding)


def kernel(tokens, embedding):
  idx_flat = tokens.astype(jnp.int32).reshape(TOTAL)
  out = _sc_gather(idx_flat, embedding)
  return out.reshape(BATCH_, SEQ_, EMB_)
